# baseline (speedup 1.0000x reference)
import functools

import jax
import jax.numpy as jnp
from jax import lax
from jax.experimental import pallas as pl
from jax.experimental.pallas import tpu as pltpu
from jax.experimental.pallas import tpu_sc as plsc

NC = 2
NS = 16
L = 16
NW = NC * NS


def _make_unpool(rows, w, r_chunk):
    outw = 4 * w
    rpw = rows // NW
    nit = rpw // r_chunk
    assert rpw * NW == rows and nit * r_chunk == rpw and nit % 4 == 0
    groups = w // L
    shift = (2 * w).bit_length() - 1

    mesh = plsc.VectorSubcoreMesh(
        core_axis_name="c", subcore_axis_name="s", num_cores=NC, num_subcores=NS
    )

    @functools.partial(
        pl.kernel,
        out_type=jax.ShapeDtypeStruct((2 * rows, 2 * w), jnp.float32),
        mesh=mesh,
        scratch_types=[
            pltpu.VMEM((r_chunk, w), jnp.float32),
            pltpu.VMEM((r_chunk, w), jnp.float32),
            pltpu.VMEM((r_chunk, w), jnp.int32),
            pltpu.VMEM((r_chunk, w), jnp.int32),
            pltpu.VMEM((r_chunk, w), jnp.int32),
            pltpu.VMEM((r_chunk, w), jnp.int32),
            pltpu.VMEM((2 * r_chunk, 2 * w), jnp.float32),
            pltpu.VMEM((2 * r_chunk, 2 * w), jnp.float32),
            pltpu.SemaphoreType.DMA,
            pltpu.SemaphoreType.DMA,
            pltpu.SemaphoreType.DMA,
            pltpu.SemaphoreType.DMA,
        ],
        compiler_params=pltpu.CompilerParams(
            use_tc_tiling_on_sc=True, needs_layout_passes=False
        ),
    )
    def unpool(x_hbm, idx_hbm, out_hbm, xv0, xv1, iv0, iv1, iv2, iv3,
               ov0, ov1, isem0, isem1, osem0, osem1):
        wid = lax.axis_index("c") * NS + lax.axis_index("s")
        base_row = wid * rpw
        xvs = (xv0, xv1)
        ivs = (iv0, iv1, iv2, iv3)
        ovs = (ov0, ov1)
        isems = (isem0, isem1)
        osems = (osem0, osem1)
        zeros = jnp.zeros((L,), jnp.float32)

        def start_in(it, islot):
            row0 = base_row + it * r_chunk
            pltpu.async_copy(x_hbm.at[pl.ds(row0, r_chunk)], xvs[islot % 2],
                             isems[islot % 2])
            pltpu.async_copy(idx_hbm.at[pl.ds(row0, r_chunk)], ivs[islot],
                             isems[islot % 2])

        def wait_in(it, islot):
            row0 = base_row + it * r_chunk
            pltpu.make_async_copy(
                x_hbm.at[pl.ds(row0, r_chunk)], xvs[islot % 2], isems[islot % 2]
            ).wait()
            pltpu.make_async_copy(
                idx_hbm.at[pl.ds(row0, r_chunk)], ivs[islot], isems[islot % 2]
            ).wait()

        def scatter_pass(iv_ref, ov_ref, x_ref):
            @plsc.parallel_loop(0, r_chunk, 1, unroll=2)
            def _(rr):
                row2 = jnp.full((L,), 2 * rr, jnp.int32)
                for gg in range(groups):
                    idxv = iv_ref[rr, pl.ds(gg * L, L)]
                    vals = zeros if x_ref is None else x_ref[rr, pl.ds(gg * L, L)]
                    local = lax.bitwise_and(idxv, jnp.int32(outw - 1))
                    rowv = row2 + lax.shift_right_logical(local, shift)
                    colv = lax.bitwise_and(local, jnp.int32(2 * w - 1))
                    plsc.store_scatter(ov_ref, [rowv, colv], vals)

        for ov in ovs:
            @plsc.parallel_loop(0, 2 * r_chunk, 1, unroll=2)
            def _(orow):
                for cc in range(2 * w // L):
                    ov[orow, pl.ds(cc * L, L)] = zeros

        start_in(0, 0)
        start_in(1, 1)

        def step(it, islot):
            oslot = islot % 2
            row0 = base_row + it * r_chunk
            wait_in(it, islot)

            @pl.when(it >= 2)
            def _():
                pltpu.make_async_copy(
                    ovs[oslot],
                    out_hbm.at[pl.ds(2 * (row0 - 2 * r_chunk), 2 * r_chunk)],
                    osems[oslot],
                ).wait()
                scatter_pass(ivs[islot - 2], ovs[oslot], None)

            scatter_pass(ivs[islot], ovs[oslot], xvs[islot % 2])

            pltpu.async_copy(
                ovs[oslot], out_hbm.at[pl.ds(2 * row0, 2 * r_chunk)], osems[oslot]
            )

            @pl.when(it + 2 < nit)
            def _():
                start_in(it + 2, (islot + 2) % 4)

        def outer(i, carry):
            for k in range(4):
                step(4 * i + k, k)
            return carry

        lax.fori_loop(0, nit // 4, outer, 0)

        last0 = base_row + (nit - 2) * r_chunk
        pltpu.make_async_copy(
            ovs[0], out_hbm.at[pl.ds(2 * last0, 2 * r_chunk)], osems[0]
        ).wait()
        pltpu.make_async_copy(
            ovs[1], out_hbm.at[pl.ds(2 * (last0 + r_chunk), 2 * r_chunk)], osems[1]
        ).wait()

    return unpool


def kernel(x, indices, output_size):
    del output_size
    B, C, H, W = x.shape
    OH, OW = 2 * H, 2 * W
    rows = B * C * H
    xf = x.reshape(rows, W)
    idxf = indices.astype(jnp.int32).reshape(rows, W)
    out = _make_unpool(rows, W, 32)(xf, idxf)
    return out.reshape(B, C, OH, OW)


# speedup vs baseline: 1.4220x; 1.4220x over previous
import functools

import jax
import jax.numpy as jnp
from jax import lax
from jax.experimental import pallas as pl
from jax.experimental.pallas import tpu as pltpu
from jax.experimental.pallas import tpu_sc as plsc

NC = 2
NS = 16
L = 16
NW = NC * NS


def _make_unpool(rows, w, r_chunk):
    outw = 4 * w
    rpw = rows // NW
    nit = rpw // r_chunk
    assert rpw * NW == rows and nit * r_chunk == rpw and nit % 2 == 0
    groups = w // L
    shift = (2 * w).bit_length() - 1

    mesh = plsc.VectorSubcoreMesh(
        core_axis_name="c", subcore_axis_name="s", num_cores=NC, num_subcores=NS
    )

    @functools.partial(
        pl.kernel,
        out_type=jax.ShapeDtypeStruct((2 * rows, 2 * w), jnp.float32),
        mesh=mesh,
        scratch_types=[
            pltpu.VMEM((r_chunk, w), jnp.float32),
            pltpu.VMEM((r_chunk, w), jnp.float32),
            pltpu.VMEM((r_chunk, w), jnp.int32),
            pltpu.VMEM((r_chunk, w), jnp.int32),
            pltpu.VMEM((2 * r_chunk, 2 * w), jnp.float32),
            pltpu.VMEM((2 * r_chunk, 2 * w), jnp.float32),
            pltpu.SemaphoreType.DMA,
            pltpu.SemaphoreType.DMA,
            pltpu.SemaphoreType.DMA,
            pltpu.SemaphoreType.DMA,
        ],
        compiler_params=pltpu.CompilerParams(
            use_tc_tiling_on_sc=True, needs_layout_passes=False
        ),
    )
    def unpool(x_hbm, idx_hbm, out_hbm, xv0, xv1, iv0, iv1, ov0, ov1,
               isem0, isem1, osem0, osem1):
        wid = lax.axis_index("c") * NS + lax.axis_index("s")
        base_row = wid * rpw
        xvs = (xv0, xv1)
        ivs = (iv0, iv1)
        ovs = (ov0, ov1)
        isems = (isem0, isem1)
        osems = (osem0, osem1)
        zeros = jnp.zeros((L,), jnp.float32)

        def start_in(it, slot):
            row0 = base_row + it * r_chunk
            pltpu.async_copy(x_hbm.at[pl.ds(row0, r_chunk)], xvs[slot], isems[slot])
            pltpu.async_copy(idx_hbm.at[pl.ds(row0, r_chunk)], ivs[slot], isems[slot])

        def wait_in(it, slot):
            row0 = base_row + it * r_chunk
            pltpu.make_async_copy(
                x_hbm.at[pl.ds(row0, r_chunk)], xvs[slot], isems[slot]
            ).wait()
            pltpu.make_async_copy(
                idx_hbm.at[pl.ds(row0, r_chunk)], ivs[slot], isems[slot]
            ).wait()

        start_in(0, 0)
        start_in(1, 1)

        def step(it, slot):
            row0 = base_row + it * r_chunk

            @pl.when(it >= 2)
            def _():
                pltpu.make_async_copy(
                    ovs[slot],
                    out_hbm.at[pl.ds(2 * (row0 - 2 * r_chunk), 2 * r_chunk)],
                    osems[slot],
                ).wait()

            @plsc.parallel_loop(0, 2 * r_chunk, 1, unroll=2)
            def _(orow):
                for cc in range(2 * w // L):
                    ovs[slot][orow, pl.ds(cc * L, L)] = zeros

            wait_in(it, slot)

            @plsc.parallel_loop(0, r_chunk, 1, unroll=2)
            def _(rr):
                row2 = jnp.full((L,), 2 * rr, jnp.int32)
                for gg in range(groups):
                    idxv = ivs[slot][rr, pl.ds(gg * L, L)]
                    vals = xvs[slot][rr, pl.ds(gg * L, L)]
                    local = lax.bitwise_and(idxv, jnp.int32(outw - 1))
                    rowv = row2 + lax.shift_right_logical(local, shift)
                    colv = lax.bitwise_and(local, jnp.int32(2 * w - 1))
                    plsc.store_scatter(ovs[slot], [rowv, colv], vals)

            pltpu.async_copy(
                ovs[slot], out_hbm.at[pl.ds(2 * row0, 2 * r_chunk)], osems[slot]
            )

            @pl.when(it + 2 < nit)
            def _():
                start_in(it + 2, slot)

        def outer(i, carry):
            step(2 * i, 0)
            step(2 * i + 1, 1)
            return carry

        lax.fori_loop(0, nit // 2, outer, 0)

        last0 = base_row + (nit - 2) * r_chunk
        pltpu.make_async_copy(
            ovs[0], out_hbm.at[pl.ds(2 * last0, 2 * r_chunk)], osems[0]
        ).wait()
        pltpu.make_async_copy(
            ovs[1], out_hbm.at[pl.ds(2 * (last0 + r_chunk), 2 * r_chunk)], osems[1]
        ).wait()

    return unpool


def kernel(x, indices, output_size):
    del output_size
    B, C, H, W = x.shape
    OH, OW = 2 * H, 2 * W
    rows = B * C * H
    xf = x.reshape(rows, W)
    idxf = indices.astype(jnp.int32).reshape(rows, W)
    out = _make_unpool(rows, W, 32)(xf, idxf)
    return out.reshape(B, C, OH, OW)
